# baseline (speedup 1.0000x reference)
import jax
import jax.numpy as jnp
from jax import lax
from jax.experimental import pallas as pl
from jax.experimental.pallas import tpu as pltpu
from jax.experimental.pallas import tpu_sc as plsc

N = 10000
E = 320000
F = 128
D = 64
NC = 2
NS = 16
CPT = D // NS
E_SC = E // NC
C = 2000
NCH = E_SC // C
NG = C // 16


def _dense_front_body(emb_ref, ws_ref, bs_ref, wa1_ref, wa2_ref, ba_ref,
                      scaled_ref, alpha_ref, beta_ref):
    scaled = jnp.dot(emb_ref[...], ws_ref[...],
                     preferred_element_type=jnp.float32) + bs_ref[...]
    scaled_ref[...] = scaled
    alpha_ref[...] = jnp.dot(scaled, wa1_ref[...],
                             preferred_element_type=jnp.float32) + ba_ref[...]
    beta_ref[...] = jnp.dot(scaled, wa2_ref[...],
                            preferred_element_type=jnp.float32)


def _dense_front(emb_mat, W_scale, b_scale, wa1, wa2, b_att):
    blk = 1000
    return pl.pallas_call(
        _dense_front_body,
        grid=(N // blk,),
        in_specs=[
            pl.BlockSpec((blk, F), lambda i: (i, 0)),
            pl.BlockSpec((F, D), lambda i: (0, 0)),
            pl.BlockSpec((1, D), lambda i: (0, 0)),
            pl.BlockSpec((D, 1), lambda i: (0, 0)),
            pl.BlockSpec((D, 1), lambda i: (0, 0)),
            pl.BlockSpec((1, 1), lambda i: (0, 0)),
        ],
        out_specs=[
            pl.BlockSpec((blk, D), lambda i: (i, 0)),
            pl.BlockSpec((blk, 1), lambda i: (i, 0)),
            pl.BlockSpec((blk, 1), lambda i: (i, 0)),
        ],
        out_shape=[
            jax.ShapeDtypeStruct((N, D), jnp.float32),
            jax.ShapeDtypeStruct((N, 1), jnp.float32),
            jax.ShapeDtypeStruct((N, 1), jnp.float32),
        ],
    )(emb_mat, W_scale, b_scale.reshape(1, D), wa1, wa2, b_att.reshape(1, 1))


def _sc_body(src_hbm, dst_hbm, alpha_hbm, beta_hbm, scol_hbm,
             acc_out, ssum_out,
             alpha_v, beta_v, scol, accc, lssum,
             src0, src1, dst0, dst1, lsem0, lsem1):
    cid = lax.axis_index("c")
    sid = lax.axis_index("s")

    zero16 = jnp.zeros((16,), jnp.float32)

    @pl.loop(0, (N * CPT) // 16)
    def _(p):
        accc[pl.ds(p * 16, 16)] = zero16

    @pl.loop(0, N // 16)
    def _(p):
        lssum[pl.ds(p * 16, 16)] = zero16

    pltpu.sync_copy(scol_hbm.at[sid], scol)
    pltpu.sync_copy(alpha_hbm, alpha_v)
    pltpu.sync_copy(beta_hbm, beta_v)

    sbufs = (src0, src1)
    dbufs = (dst0, dst1)
    lsems = (lsem0, lsem1)
    ebase = cid * E_SC

    def start_chunk(ch, k):
        pltpu.async_copy(src_hbm.at[pl.ds(ebase + ch * C, C)], sbufs[k],
                         lsems[k])
        pltpu.async_copy(dst_hbm.at[pl.ds(ebase + ch * C, C)], dbufs[k],
                         lsems[k])

    def wait_chunk(ch, k):
        pltpu.make_async_copy(src_hbm.at[pl.ds(ebase + ch * C, C)], sbufs[k],
                              lsems[k]).wait()
        pltpu.make_async_copy(dst_hbm.at[pl.ds(ebase + ch * C, C)], dbufs[k],
                              lsems[k]).wait()

    for k in range(2):
        start_chunk(k, k)

    @pl.loop(0, NCH, step=2)
    def _(ch):
        for k in range(2):
            wait_chunk(ch + k, k)

            @pl.loop(0, NG)
            def _(g):
                src16 = sbufs[k][pl.ds(g * 16, 16)]
                dst16 = dbufs[k][pl.ds(g * 16, 16)]
                att = (plsc.load_gather(alpha_v, [src16])
                       + plsc.load_gather(beta_v, [dst16]))
                att = jnp.where(att >= 0.0, att, 0.2 * att)
                s = jnp.exp(att - 1.0)
                plsc.addupdate_scatter(lssum, [src16], s)
                sidx = src16 * CPT
                didx = dst16 * CPT
                for c in range(CPT):
                    v = plsc.load_gather(scol, [didx + c]) * s
                    plsc.addupdate_scatter(accc, [sidx + c], v)

            @pl.when(ch + k + 2 < NCH)
            def _():
                start_chunk(ch + k + 2, k)

    pltpu.sync_copy(accc, acc_out.at[cid, sid])

    @pl.when(sid < 10)
    def _():
        pltpu.sync_copy(lssum.at[pl.ds(sid * 1000, 1000)],
                        ssum_out.at[cid, sid])


def _sc_aggregate(src, dst, alpha, beta, scolT):
    mesh = plsc.VectorSubcoreMesh(core_axis_name="c", subcore_axis_name="s",
                                  num_cores=NC, num_subcores=NS)
    kern = pl.kernel(
        _sc_body,
        out_type=(
            jax.ShapeDtypeStruct((NC, NS, N * CPT), jnp.float32),
            jax.ShapeDtypeStruct((NC, 10, 1000), jnp.float32),
        ),
        mesh=mesh,
        compiler_params=pltpu.CompilerParams(needs_layout_passes=False,
                                             use_tc_tiling_on_sc=False),
        scratch_types=[
            pltpu.VMEM((N,), jnp.float32),
            pltpu.VMEM((N,), jnp.float32),
            pltpu.VMEM((N * CPT,), jnp.float32),
            pltpu.VMEM((N * CPT,), jnp.float32),
            pltpu.VMEM((N,), jnp.float32),
            pltpu.VMEM((C,), jnp.int32),
            pltpu.VMEM((C,), jnp.int32),
            pltpu.VMEM((C,), jnp.int32),
            pltpu.VMEM((C,), jnp.int32),
            pltpu.SemaphoreType.DMA,
            pltpu.SemaphoreType.DMA,
        ],
    )
    return kern(src, dst, alpha, beta, scolT)


def _final_body(acc0_ref, acc1_ref, ssumt_ref, out_ref):
    a = acc0_ref[...] + acc1_ref[...]
    ss = ssumt_ref[:, 0:1] + ssumt_ref[:, 1:2]
    ss = jnp.where(ss == 0.0, 1.0, ss)
    x = a / ss
    out_ref[...] = 1.0 / (1.0 + jnp.exp(-x))


def _final(acc0, acc1, ssumt):
    blk = 1000
    return pl.pallas_call(
        _final_body,
        grid=(N // blk,),
        in_specs=[
            pl.BlockSpec((blk, D), lambda i: (i, 0)),
            pl.BlockSpec((blk, D), lambda i: (i, 0)),
            pl.BlockSpec((blk, 2), lambda i: (i, 0)),
        ],
        out_specs=pl.BlockSpec((blk, D), lambda i: (i, 0)),
        out_shape=jax.ShapeDtypeStruct((N, D), jnp.float32),
    )(acc0, acc1, ssumt)


def kernel(emb_mat, edge, W_scale, b_scale, W_att, b_att):
    src = edge[:, 0]
    dst = edge[:, 1]
    wa1 = W_att[:D]
    wa2 = W_att[D:]
    scaled, alpha, beta = _dense_front(emb_mat, W_scale, b_scale, wa1, wa2,
                                       b_att)
    scolT = scaled.reshape(N, NS, CPT).transpose(1, 0, 2).reshape(NS, N * CPT)
    acc, ssum = _sc_aggregate(src, dst, alpha.reshape(N), beta.reshape(N),
                              scolT)
    accf = (acc.reshape(NC, NS, N, CPT).transpose(0, 2, 1, 3)
            .reshape(NC, N, D))
    ssumt = ssum.reshape(NC, N).T
    return _final(accf[0], accf[1], ssumt)


# speedup vs baseline: 1.3694x; 1.3694x over previous
import jax
import jax.numpy as jnp
from jax import lax
from jax.experimental import pallas as pl
from jax.experimental.pallas import tpu as pltpu
from jax.experimental.pallas import tpu_sc as plsc

N = 10000
E = 320000
F = 128
D = 64
NC = 2
NS = 16
CPT = D // NS
E_SC = E // NC
C = 2000
NCH = E_SC // C
NG = C // 16


def _dense_front_body(emb_ref, ws_ref, bs_ref, wa1_ref, wa2_ref, ba_ref,
                      scaled_ref, alpha_ref, beta_ref):
    scaled = jnp.dot(emb_ref[...], ws_ref[...],
                     preferred_element_type=jnp.float32) + bs_ref[...]
    scaled_ref[...] = scaled
    alpha_ref[...] = jnp.dot(scaled, wa1_ref[...],
                             preferred_element_type=jnp.float32) + ba_ref[...]
    beta_ref[...] = jnp.dot(scaled, wa2_ref[...],
                            preferred_element_type=jnp.float32)


def _dense_front(emb_mat, W_scale, b_scale, wa1, wa2, b_att):
    blk = 1000
    return pl.pallas_call(
        _dense_front_body,
        grid=(N // blk,),
        in_specs=[
            pl.BlockSpec((blk, F), lambda i: (i, 0)),
            pl.BlockSpec((F, D), lambda i: (0, 0)),
            pl.BlockSpec((1, D), lambda i: (0, 0)),
            pl.BlockSpec((D, 1), lambda i: (0, 0)),
            pl.BlockSpec((D, 1), lambda i: (0, 0)),
            pl.BlockSpec((1, 1), lambda i: (0, 0)),
        ],
        out_specs=[
            pl.BlockSpec((blk, D), lambda i: (i, 0)),
            pl.BlockSpec((blk, 1), lambda i: (i, 0)),
            pl.BlockSpec((blk, 1), lambda i: (i, 0)),
        ],
        out_shape=[
            jax.ShapeDtypeStruct((N, D), jnp.float32),
            jax.ShapeDtypeStruct((N, 1), jnp.float32),
            jax.ShapeDtypeStruct((N, 1), jnp.float32),
        ],
    )(emb_mat, W_scale, b_scale.reshape(1, D), wa1, wa2, b_att.reshape(1, 1))


def _sc_body(src_hbm, dst_hbm, alpha_hbm, beta_hbm, scol_hbm,
             acc_out, ssum_out,
             alpha_v, beta_v, scol, accc, lssum,
             src0, src1, dst0, dst1, lsem0, lsem1):
    cid = lax.axis_index("c")
    sid = lax.axis_index("s")

    zero16 = jnp.zeros((16,), jnp.float32)

    @pl.loop(0, (N * CPT) // 16)
    def _(p):
        accc[pl.ds(p * 16, 16)] = zero16

    @pl.loop(0, N // 16)
    def _(p):
        lssum[pl.ds(p * 16, 16)] = zero16

    pltpu.sync_copy(scol_hbm.at[sid], scol)
    pltpu.sync_copy(alpha_hbm, alpha_v)
    pltpu.sync_copy(beta_hbm, beta_v)

    sbufs = (src0, src1)
    dbufs = (dst0, dst1)
    lsems = (lsem0, lsem1)
    ebase = cid * E_SC

    def start_chunk(ch, k):
        pltpu.async_copy(src_hbm.at[pl.ds(ebase + ch * C, C)], sbufs[k],
                         lsems[k])
        pltpu.async_copy(dst_hbm.at[pl.ds(ebase + ch * C, C)], dbufs[k],
                         lsems[k])

    def wait_chunk(ch, k):
        pltpu.make_async_copy(src_hbm.at[pl.ds(ebase + ch * C, C)], sbufs[k],
                              lsems[k]).wait()
        pltpu.make_async_copy(dst_hbm.at[pl.ds(ebase + ch * C, C)], dbufs[k],
                              lsems[k]).wait()

    for k in range(2):
        start_chunk(k, k)

    @pl.loop(0, NCH, step=2)
    def _(ch):
        for k in range(2):
            wait_chunk(ch + k, k)

            @plsc.parallel_loop(0, NG, unroll=4)
            def _(g):
                src16 = sbufs[k][pl.ds(g * 16, 16)]
                dst16 = dbufs[k][pl.ds(g * 16, 16)]
                att = (plsc.load_gather(alpha_v, [src16])
                       + plsc.load_gather(beta_v, [dst16]))
                att = jnp.where(att >= 0.0, att, 0.2 * att)
                s = jnp.exp(att - 1.0)
                plsc.addupdate_scatter(lssum, [src16], s)
                sidx = src16 * CPT
                didx = dst16 * CPT
                for c in range(CPT):
                    v = plsc.load_gather(scol, [didx + c]) * s
                    plsc.addupdate_scatter(accc, [sidx + c], v)

            @pl.when(ch + k + 2 < NCH)
            def _():
                start_chunk(ch + k + 2, k)

    pltpu.sync_copy(accc, acc_out.at[cid, sid])

    @pl.when(sid < 10)
    def _():
        pltpu.sync_copy(lssum.at[pl.ds(sid * 1000, 1000)],
                        ssum_out.at[cid, sid])


def _sc_aggregate(src, dst, alpha, beta, scolT):
    mesh = plsc.VectorSubcoreMesh(core_axis_name="c", subcore_axis_name="s",
                                  num_cores=NC, num_subcores=NS)
    kern = pl.kernel(
        _sc_body,
        out_type=(
            jax.ShapeDtypeStruct((NC, NS, N * CPT), jnp.float32),
            jax.ShapeDtypeStruct((NC, 10, 1000), jnp.float32),
        ),
        mesh=mesh,
        compiler_params=pltpu.CompilerParams(needs_layout_passes=False,
                                             use_tc_tiling_on_sc=False),
        scratch_types=[
            pltpu.VMEM((N,), jnp.float32),
            pltpu.VMEM((N,), jnp.float32),
            pltpu.VMEM((N * CPT,), jnp.float32),
            pltpu.VMEM((N * CPT,), jnp.float32),
            pltpu.VMEM((N,), jnp.float32),
            pltpu.VMEM((C,), jnp.int32),
            pltpu.VMEM((C,), jnp.int32),
            pltpu.VMEM((C,), jnp.int32),
            pltpu.VMEM((C,), jnp.int32),
            pltpu.SemaphoreType.DMA,
            pltpu.SemaphoreType.DMA,
        ],
    )
    return kern(src, dst, alpha, beta, scolT)


def _final_body(acc0_ref, acc1_ref, ssumt_ref, out_ref):
    a = acc0_ref[...] + acc1_ref[...]
    ss = ssumt_ref[:, 0:1] + ssumt_ref[:, 1:2]
    ss = jnp.where(ss == 0.0, 1.0, ss)
    x = a / ss
    out_ref[...] = 1.0 / (1.0 + jnp.exp(-x))


def _final(acc0, acc1, ssumt):
    blk = 1000
    return pl.pallas_call(
        _final_body,
        grid=(N // blk,),
        in_specs=[
            pl.BlockSpec((blk, D), lambda i: (i, 0)),
            pl.BlockSpec((blk, D), lambda i: (i, 0)),
            pl.BlockSpec((blk, 2), lambda i: (i, 0)),
        ],
        out_specs=pl.BlockSpec((blk, D), lambda i: (i, 0)),
        out_shape=jax.ShapeDtypeStruct((N, D), jnp.float32),
    )(acc0, acc1, ssumt)


def kernel(emb_mat, edge, W_scale, b_scale, W_att, b_att):
    src = edge[:, 0]
    dst = edge[:, 1]
    wa1 = W_att[:D]
    wa2 = W_att[D:]
    scaled, alpha, beta = _dense_front(emb_mat, W_scale, b_scale, wa1, wa2,
                                       b_att)
    scolT = scaled.reshape(N, NS, CPT).transpose(1, 0, 2).reshape(NS, N * CPT)
    acc, ssum = _sc_aggregate(src, dst, alpha.reshape(N), beta.reshape(N),
                              scolT)
    accf = (acc.reshape(NC, NS, N, CPT).transpose(0, 2, 1, 3)
            .reshape(NC, N, D))
    ssumt = ssum.reshape(NC, N).T
    return _final(accf[0], accf[1], ssumt)


# speedup vs baseline: 3.9099x; 2.8551x over previous
import jax
import jax.numpy as jnp
from jax import lax
from jax.experimental import pallas as pl
from jax.experimental.pallas import tpu as pltpu
from jax.experimental.pallas import tpu_sc as plsc

N = 10000
E = 320000
F = 128
D = 64
NC = 2
NS = 16
NW = NC * NS
PER_W = E // NW
B = 128
NSUB = 80
PER_W_PAD = NSUB * B


def _dense_front_body(emb_ref, ws_ref, bs_ref, wa1_ref, wa2_ref, ba_ref,
                      scaled_ref, alpha_ref, beta_ref):
    scaled = jnp.dot(emb_ref[...], ws_ref[...],
                     preferred_element_type=jnp.float32) + bs_ref[...]
    scaled_ref[...] = scaled
    alpha_ref[...] = jnp.dot(scaled, wa1_ref[...],
                             preferred_element_type=jnp.float32) + ba_ref[...]
    beta_ref[...] = jnp.dot(scaled, wa2_ref[...],
                            preferred_element_type=jnp.float32)


def _dense_front(emb_mat, W_scale, b_scale, wa1, wa2, b_att):
    blk = 1000
    return pl.pallas_call(
        _dense_front_body,
        grid=(N // blk,),
        in_specs=[
            pl.BlockSpec((blk, F), lambda i: (i, 0)),
            pl.BlockSpec((F, D), lambda i: (0, 0)),
            pl.BlockSpec((1, D), lambda i: (0, 0)),
            pl.BlockSpec((D, 1), lambda i: (0, 0)),
            pl.BlockSpec((D, 1), lambda i: (0, 0)),
            pl.BlockSpec((1, 1), lambda i: (0, 0)),
        ],
        out_specs=[
            pl.BlockSpec((blk, D), lambda i: (i, 0)),
            pl.BlockSpec((blk, 1), lambda i: (i, 0)),
            pl.BlockSpec((blk, 1), lambda i: (i, 0)),
        ],
        out_shape=[
            jax.ShapeDtypeStruct((N, D), jnp.float32),
            jax.ShapeDtypeStruct((N, 1), jnp.float32),
            jax.ShapeDtypeStruct((N, 1), jnp.float32),
        ],
    )(emb_mat, W_scale, b_scale.reshape(1, D), wa1, wa2, b_att.reshape(1, 1))


def _sc_body(src_hbm, dst_hbm, alpha_hbm, beta_hbm, scaled_hbm,
             acc_out, ssum_out,
             srci, dsti, alpha_v, beta_v, svals, rows0, rows1,
             fbuf0, fbuf1, zrow, zflat, acc_sh, ssum_sh,
             gsem0, gsem1, ssem0, ssem1, sem_s):
    cid = lax.axis_index("c")
    sid = lax.axis_index("s")
    wid = sid * NC + cid

    zero16 = jnp.zeros((16,), jnp.float32)

    @pl.loop(0, (125 * D) // 16)
    def _(p):
        r = p // 4
        c = (p % 4) * 16
        zrow[r, pl.ds(c, 16)] = zero16

    @pl.loop(0, 1024 // 16)
    def _(p):
        zflat[pl.ds(p * 16, 16)] = zero16

    @pl.when(sid < 10)
    def _():
        @pl.loop(0, 8)
        def _(k):
            pltpu.sync_copy(zrow,
                            acc_sh.at[pl.ds(sid * 1000 + k * 125, 125), :])

        pltpu.sync_copy(zflat.at[pl.ds(0, 1000)],
                        ssum_sh.at[pl.ds(sid * 1000, 1000)])

    pltpu.sync_copy(src_hbm.at[wid], srci)
    pltpu.sync_copy(dst_hbm.at[wid], dsti)
    pltpu.sync_copy(alpha_hbm, alpha_v)
    pltpu.sync_copy(beta_hbm, beta_v)
    plsc.subcore_barrier()

    lanes = lax.iota(jnp.int32, 16)

    def score_batch(j):
        for v in range(B // 16):
            src16 = srci[j, pl.ds(v * 16, 16)]
            dst16 = dsti[j, pl.ds(v * 16, 16)]
            att = (plsc.load_gather(alpha_v, [src16])
                   + plsc.load_gather(beta_v, [dst16]))
            att = jnp.where(att >= 0.0, att, 0.2 * att)
            s = jnp.exp(att - 1.0)
            pos = j * B + v * 16 + lanes
            s = jnp.where(pos < PER_W, s, 0.0)
            svals[j, pl.ds(v * 16, 16)] = s

    def mul_rows(rows, fbuf, j):
        bj = jnp.zeros((16,), jnp.int32) + j

        @pl.loop(0, B, unroll=4)
        def _(r):
            w = plsc.load_gather(svals, [bj, jnp.zeros((16,), jnp.int32) + r])
            for g in range(D // 32):
                x = rows[r, pl.ds(g * 32, 32)]
                a, b = plsc.unpack(x, format=plsc.PackFormat.INTERLEAVED)
                fbuf[r, pl.ds(g * 32, 16)] = a * w
                fbuf[r, pl.ds(g * 32 + 16, 16)] = b * w

    rbufs = (rows0, rows1)
    gsems = (gsem0, gsem1)
    fbufs = (fbuf0, fbuf1)
    ssems = (ssem0, ssem1)

    for k in range(2):
        pltpu.async_copy(scaled_hbm.at[dsti.at[k]], rbufs[k], gsems[k])
        score_batch(k)

    @pl.loop(0, NSUB, step=2)
    def _(j):
        @pl.when(j > 0)
        def _():
            for _k in range(2):
                pltpu.make_async_copy(svals.at[j], ssum_sh.at[srci.at[j]],
                                      sem_s).wait()

        for k in range(2):
            pltpu.async_copy(svals.at[j + k], ssum_sh.at[srci.at[j + k]],
                             sem_s, add=True)

        for k in range(2):
            fb = fbufs[k]
            pltpu.make_async_copy(scaled_hbm.at[dsti.at[j + k]], rbufs[k],
                                  gsems[k]).wait()

            @pl.when(j > 0)
            def _():
                pltpu.make_async_copy(fb, acc_sh.at[srci.at[j]],
                                      ssems[k]).wait()

            @pl.when(j + k + 2 < NSUB)
            def _():
                score_batch(j + k + 2)

            mul_rows(rbufs[k], fb, j + k)
            @pl.when(j + k + 2 < NSUB)
            def _():
                pltpu.async_copy(scaled_hbm.at[dsti.at[j + k + 2]], rbufs[k],
                                 gsems[k])

            pltpu.async_copy(fb, acc_sh.at[srci.at[j + k]], ssems[k],
                             add=True)

    for k in range(2):
        pltpu.make_async_copy(fbufs[k], acc_sh.at[srci.at[0]],
                              ssems[k]).wait()
    for _k in range(2):
        pltpu.make_async_copy(svals.at[0], ssum_sh.at[srci.at[0]],
                              sem_s).wait()

    plsc.subcore_barrier()

    @pl.when(sid < 10)
    def _():
        pltpu.sync_copy(acc_sh.at[pl.ds(sid * 1000, 1000), :],
                        acc_out.at[cid, pl.ds(sid * 1000, 1000), :])
        pltpu.sync_copy(ssum_sh.at[pl.ds(sid * 1000, 1000)],
                        ssum_out.at[pl.ds(cid * N + sid * 1000, 1000)])


def _sc_aggregate(src3, dst3, alpha, beta, scaled):
    mesh = plsc.VectorSubcoreMesh(core_axis_name="c", subcore_axis_name="s",
                                  num_cores=NC, num_subcores=NS)
    kern = pl.kernel(
        _sc_body,
        out_type=(
            jax.ShapeDtypeStruct((NC, N, D), jnp.float32),
            jax.ShapeDtypeStruct((NC * N,), jnp.float32),
        ),
        mesh=mesh,
        compiler_params=pltpu.CompilerParams(needs_layout_passes=False,
                                             use_tc_tiling_on_sc=False),
        scratch_types=[
            pltpu.VMEM((NSUB, B), jnp.int32),
            pltpu.VMEM((NSUB, B), jnp.int32),
            pltpu.VMEM((N,), jnp.float32),
            pltpu.VMEM((N,), jnp.float32),
            pltpu.VMEM((NSUB, B), jnp.float32),
            pltpu.VMEM((B, D), jnp.bfloat16),
            pltpu.VMEM((B, D), jnp.bfloat16),
            pltpu.VMEM((B, D), jnp.float32),
            pltpu.VMEM((B, D), jnp.float32),
            pltpu.VMEM((125, D), jnp.float32),
            pltpu.VMEM((1024,), jnp.float32),
            pltpu.VMEM_SHARED((N, D), jnp.float32),
            pltpu.VMEM_SHARED((N,), jnp.float32),
            pltpu.SemaphoreType.DMA,
            pltpu.SemaphoreType.DMA,
            pltpu.SemaphoreType.DMA,
            pltpu.SemaphoreType.DMA,
            pltpu.SemaphoreType.DMA,
        ],
    )
    return kern(src3, dst3, alpha, beta, scaled)


def _final_body(acc0_ref, acc1_ref, ssumt_ref, out_ref):
    a = acc0_ref[...] + acc1_ref[...]
    ss = ssumt_ref[:, 0:1] + ssumt_ref[:, 1:2]
    ss = jnp.where(ss == 0.0, 1.0, ss)
    x = a / ss
    out_ref[...] = 1.0 / (1.0 + jnp.exp(-x))


def _final(acc, ssum):
    blk = 1000
    ssumt = ssum.T
    return pl.pallas_call(
        _final_body,
        grid=(N // blk,),
        in_specs=[
            pl.BlockSpec((blk, D), lambda i: (i, 0)),
            pl.BlockSpec((blk, D), lambda i: (i, 0)),
            pl.BlockSpec((blk, 2), lambda i: (i, 0)),
        ],
        out_specs=pl.BlockSpec((blk, D), lambda i: (i, 0)),
        out_shape=jax.ShapeDtypeStruct((N, D), jnp.float32),
    )(acc[0], acc[1], ssumt)


def kernel(emb_mat, edge, W_scale, b_scale, W_att, b_att):
    src = edge[:, 0]
    dst = edge[:, 1]
    pad = PER_W_PAD - PER_W
    src3 = jnp.pad(src.reshape(NW, PER_W), ((0, 0), (0, pad))).reshape(NW, NSUB, B)
    dst3 = jnp.pad(dst.reshape(NW, PER_W), ((0, 0), (0, pad))).reshape(NW, NSUB, B)
    wa1 = W_att[:D]
    wa2 = W_att[D:]
    scaled, alpha, beta = _dense_front(emb_mat, W_scale, b_scale, wa1, wa2, b_att)
    scaled_bf = (scaled.reshape(N, D // 32, 2, 16).swapaxes(2, 3)
                 .reshape(N, D).astype(jnp.bfloat16))
    acc, ssum = _sc_aggregate(src3, dst3, alpha.reshape(N), beta.reshape(N),
                              scaled_bf)
    return _final(acc, ssum.reshape(NC, N))
